# baseline (speedup 1.0000x reference)
import functools

import jax
import jax.numpy as jnp
from jax import lax
from jax.experimental import pallas as pl
from jax.experimental.pallas import tpu as pltpu
from jax.experimental.pallas import tpu_sc as plsc

B = 16384
D = 256
K = 8192
BB = 256


def _dist_body(x_ref, lm_ref, nd_ref, idx_ref, c2_ref):
    @pl.when(pl.program_id(0) == 0)
    def _():
        lm = lm_ref[...]
        c2_ref[...] = jnp.sum(lm * lm, axis=1)[None, :]

    xb = x_ref[...]
    xb2 = xb + xb
    x2 = jnp.sum(xb * xb, axis=1)
    mm2 = lax.dot_general(xb2, lm_ref[...], (((1,), (1,)), ((), ())),
                          preferred_element_type=jnp.float32)
    t = x2[:, None] + c2_ref[...]
    nd = mm2 - t
    nd_ref[...] = nd
    ndmax = jnp.max(nd, axis=1)
    iota = lax.broadcasted_iota(jnp.int32, (BB, K), 1).astype(jnp.float32)
    idx_f = jnp.min(jnp.where(nd >= ndmax[:, None], iota, jnp.float32(K)),
                    axis=1)
    idx_ref[...] = idx_f.astype(jnp.int32)[None, None, :]


def _distances_and_argmin(x, label_mat):
    grid = (B // BB,)
    nd, idx3 = pl.pallas_call(
        _dist_body,
        grid=grid,
        in_specs=[
            pl.BlockSpec((BB, D), lambda i: (i, 0)),
            pl.BlockSpec((K, D), lambda i: (0, 0)),
        ],
        out_specs=[
            pl.BlockSpec((BB, K), lambda i: (i, 0)),
            pl.BlockSpec((1, 1, BB), lambda i: (i, 0, 0)),
        ],
        out_shape=[
            jax.ShapeDtypeStruct((B, K), jnp.float32),
            jax.ShapeDtypeStruct((B // BB, 1, BB), jnp.int32),
        ],
        scratch_shapes=[pltpu.VMEM((1, K), jnp.float32)],
        compiler_params=pltpu.CompilerParams(
            dimension_semantics=("arbitrary",),
        ),
    )(x, label_mat)
    return nd, idx3.reshape(B)


_NC = 2
_NS = 16
_NW = _NC * _NS
_BPW = B // _NW
_CHUNK = 128


def _gather_body(lm_hbm, idx_hbm, out_hbm, idx_v, rows0, rows1, gsem,
                 wsem0, wsem1):
    wid = lax.axis_index("s") * _NC + lax.axis_index("c")
    base = wid * _BPW
    pltpu.sync_copy(idx_hbm.at[pl.ds(base, _BPW)], idx_v)
    rows = (rows0, rows1)
    wsems = (wsem0, wsem1)
    nchunks = _BPW // _CHUNK
    wb = [None, None]
    for t in range(nchunks):
        b = t % 2
        if wb[b] is not None:
            wb[b].wait()
        pltpu.async_copy(
            lm_hbm.at[idx_v.at[pl.ds(t * _CHUNK, _CHUNK)]], rows[b], gsem,
        ).wait()
        wb[b] = pltpu.async_copy(
            rows[b], out_hbm.at[pl.ds(base + t * _CHUNK, _CHUNK)], wsems[b])
    for b in range(2):
        if wb[b] is not None:
            wb[b].wait()


@functools.cache
def _sc_gather():
    return pl.kernel(
        _gather_body,
        out_type=jax.ShapeDtypeStruct((B, D), jnp.float32),
        mesh=plsc.VectorSubcoreMesh(core_axis_name="c", subcore_axis_name="s",
                                    num_cores=_NC, num_subcores=_NS),
        scratch_types=[
            pltpu.VMEM((_BPW,), jnp.int32),
            pltpu.VMEM((_CHUNK, D), jnp.float32),
            pltpu.VMEM((_CHUNK, D), jnp.float32),
            pltpu.SemaphoreType.DMA,
            pltpu.SemaphoreType.DMA,
            pltpu.SemaphoreType.DMA,
        ],
    )


def kernel(x, var, label_mat):
    del var
    neg_dis, idx = _distances_and_argmin(x, label_mat)
    quantized = _sc_gather()(label_mat, idx)
    return quantized, neg_dis


# speedup vs baseline: 1.0090x; 1.0090x over previous
import functools

import jax
import jax.numpy as jnp
from jax import lax
from jax.experimental import pallas as pl
from jax.experimental.pallas import tpu as pltpu
from jax.experimental.pallas import tpu_sc as plsc

B = 16384
D = 256
K = 8192
BB = 256


def _dist_body(x_ref, lm_ref, nd_ref, idx_ref, c2_ref):
    @pl.when(pl.program_id(0) == 0)
    def _():
        lm = lm_ref[...]
        c2_ref[...] = jnp.sum(lm * lm, axis=1)[None, :]

    xb = x_ref[...]
    xb2 = xb + xb
    x2 = jnp.sum(xb * xb, axis=1)
    mm2 = lax.dot_general(xb2, lm_ref[...], (((1,), (1,)), ((), ())),
                          preferred_element_type=jnp.float32)
    t = x2[:, None] + c2_ref[...]
    nd = mm2 - t
    nd_ref[...] = nd
    ndmax = jnp.max(nd, axis=1)
    iota = lax.broadcasted_iota(jnp.int32, (BB, K), 1).astype(jnp.float32)
    idx_f = jnp.min(jnp.where(nd >= ndmax[:, None], iota, jnp.float32(K)),
                    axis=1)
    idx_ref[...] = idx_f.astype(jnp.int32)[None, None, :]


def _distances_and_argmin(x, label_mat):
    grid = (B // BB,)
    nd, idx3 = pl.pallas_call(
        _dist_body,
        grid=grid,
        in_specs=[
            pl.BlockSpec((BB, D), lambda i: (i, 0)),
            pl.BlockSpec((K, D), lambda i: (0, 0)),
        ],
        out_specs=[
            pl.BlockSpec((BB, K), lambda i: (i, 0)),
            pl.BlockSpec((1, 1, BB), lambda i: (i, 0, 0)),
        ],
        out_shape=[
            jax.ShapeDtypeStruct((B, K), jnp.float32),
            jax.ShapeDtypeStruct((B // BB, 1, BB), jnp.int32),
        ],
        scratch_shapes=[pltpu.VMEM((1, K), jnp.float32)],
        compiler_params=pltpu.CompilerParams(
            dimension_semantics=("arbitrary",),
        ),
    )(x, label_mat)
    return nd, idx3.reshape(B)


_NC = 2
_NS = 16
_NW = _NC * _NS
_BPW = B // _NW
_CHUNK = 128


_NBUF = 3


def _gather_body(lm_hbm, idx_hbm, out_hbm, idx_v, rows0, rows1, rows2,
                 gsem0, gsem1, gsem2, wsem0, wsem1, wsem2):
    wid = lax.axis_index("s") * _NC + lax.axis_index("c")
    base = wid * _BPW
    pltpu.sync_copy(idx_hbm.at[pl.ds(base, _BPW)], idx_v)
    rows = (rows0, rows1, rows2)
    gsems = (gsem0, gsem1, gsem2)
    wsems = (wsem0, wsem1, wsem2)
    nchunks = _BPW // _CHUNK
    g = [None] * _NBUF
    wb = [None] * _NBUF

    def start_gather(t):
        b = t % _NBUF
        if wb[b] is not None:
            wb[b].wait()
        g[b] = pltpu.async_copy(
            lm_hbm.at[idx_v.at[pl.ds(t * _CHUNK, _CHUNK)]], rows[b], gsems[b])

    for t in range(min(_NBUF, nchunks)):
        start_gather(t)
    for t in range(nchunks):
        b = t % _NBUF
        g[b].wait()
        wb[b] = pltpu.async_copy(
            rows[b], out_hbm.at[pl.ds(base + t * _CHUNK, _CHUNK)], wsems[b])
        if t + _NBUF < nchunks:
            start_gather(t + _NBUF)
    for b in range(_NBUF):
        if wb[b] is not None:
            wb[b].wait()


@functools.cache
def _sc_gather():
    return pl.kernel(
        _gather_body,
        out_type=jax.ShapeDtypeStruct((B, D), jnp.float32),
        mesh=plsc.VectorSubcoreMesh(core_axis_name="c", subcore_axis_name="s",
                                    num_cores=_NC, num_subcores=_NS),
        scratch_types=[
            pltpu.VMEM((_BPW,), jnp.int32),
            pltpu.VMEM((_CHUNK, D), jnp.float32),
            pltpu.VMEM((_CHUNK, D), jnp.float32),
            pltpu.VMEM((_CHUNK, D), jnp.float32),
            pltpu.SemaphoreType.DMA,
            pltpu.SemaphoreType.DMA,
            pltpu.SemaphoreType.DMA,
            pltpu.SemaphoreType.DMA,
            pltpu.SemaphoreType.DMA,
            pltpu.SemaphoreType.DMA,
        ],
    )


def kernel(x, var, label_mat):
    del var
    neg_dis, idx = _distances_and_argmin(x, label_mat)
    quantized = _sc_gather()(label_mat, idx)
    return quantized, neg_dis


# speedup vs baseline: 1.0752x; 1.0656x over previous
import functools

import jax
import jax.numpy as jnp
from jax import lax
from jax.experimental import pallas as pl
from jax.experimental.pallas import tpu as pltpu
from jax.experimental.pallas import tpu_sc as plsc

B = 16384
D = 256
K = 8192
BB = 512


def _dist_body(x_ref, lm_ref, nd_ref, idx_ref, c2_ref):
    @pl.when(pl.program_id(0) == 0)
    def _():
        lm = lm_ref[...]
        c2_ref[...] = jnp.sum(lm * lm, axis=1)[None, :]

    xb = x_ref[...]
    xb2 = xb + xb
    x2 = jnp.sum(xb * xb, axis=1)
    mm2 = lax.dot_general(xb2, lm_ref[...], (((1,), (1,)), ((), ())),
                          preferred_element_type=jnp.float32)
    t = x2[:, None] + c2_ref[...]
    nd = mm2 - t
    nd_ref[...] = nd
    ndmax = jnp.max(nd, axis=1)
    iota = lax.broadcasted_iota(jnp.int32, (BB, K), 1).astype(jnp.float32)
    idx_f = jnp.min(jnp.where(nd >= ndmax[:, None], iota, jnp.float32(K)),
                    axis=1)
    idx_ref[...] = idx_f.astype(jnp.int32)[None, None, :]


def _distances_and_argmin(x, label_mat):
    grid = (B // BB,)
    nd, idx3 = pl.pallas_call(
        _dist_body,
        grid=grid,
        in_specs=[
            pl.BlockSpec((BB, D), lambda i: (i, 0)),
            pl.BlockSpec((K, D), lambda i: (0, 0)),
        ],
        out_specs=[
            pl.BlockSpec((BB, K), lambda i: (i, 0)),
            pl.BlockSpec((1, 1, BB), lambda i: (i, 0, 0)),
        ],
        out_shape=[
            jax.ShapeDtypeStruct((B, K), jnp.float32),
            jax.ShapeDtypeStruct((B // BB, 1, BB), jnp.int32),
        ],
        scratch_shapes=[pltpu.VMEM((1, K), jnp.float32)],
        compiler_params=pltpu.CompilerParams(
            dimension_semantics=("arbitrary",),
        ),
    )(x, label_mat)
    return nd, idx3.reshape(B)


_NC = 2
_NS = 16
_NW = _NC * _NS
_BPW = B // _NW
_CHUNK = 128


_NBUF = 3


def _gather_body(lm_hbm, idx_hbm, out_hbm, idx_v, rows0, rows1, rows2,
                 gsem0, gsem1, gsem2, wsem0, wsem1, wsem2):
    wid = lax.axis_index("s") * _NC + lax.axis_index("c")
    base = wid * _BPW
    pltpu.sync_copy(idx_hbm.at[pl.ds(base, _BPW)], idx_v)
    rows = (rows0, rows1, rows2)
    gsems = (gsem0, gsem1, gsem2)
    wsems = (wsem0, wsem1, wsem2)
    nchunks = _BPW // _CHUNK
    g = [None] * _NBUF
    wb = [None] * _NBUF

    def start_gather(t):
        b = t % _NBUF
        if wb[b] is not None:
            wb[b].wait()
        g[b] = pltpu.async_copy(
            lm_hbm.at[idx_v.at[pl.ds(t * _CHUNK, _CHUNK)]], rows[b], gsems[b])

    for t in range(min(_NBUF, nchunks)):
        start_gather(t)
    for t in range(nchunks):
        b = t % _NBUF
        g[b].wait()
        wb[b] = pltpu.async_copy(
            rows[b], out_hbm.at[pl.ds(base + t * _CHUNK, _CHUNK)], wsems[b])
        if t + _NBUF < nchunks:
            start_gather(t + _NBUF)
    for b in range(_NBUF):
        if wb[b] is not None:
            wb[b].wait()


@functools.cache
def _sc_gather():
    return pl.kernel(
        _gather_body,
        out_type=jax.ShapeDtypeStruct((B, D), jnp.float32),
        mesh=plsc.VectorSubcoreMesh(core_axis_name="c", subcore_axis_name="s",
                                    num_cores=_NC, num_subcores=_NS),
        scratch_types=[
            pltpu.VMEM((_BPW,), jnp.int32),
            pltpu.VMEM((_CHUNK, D), jnp.float32),
            pltpu.VMEM((_CHUNK, D), jnp.float32),
            pltpu.VMEM((_CHUNK, D), jnp.float32),
            pltpu.SemaphoreType.DMA,
            pltpu.SemaphoreType.DMA,
            pltpu.SemaphoreType.DMA,
            pltpu.SemaphoreType.DMA,
            pltpu.SemaphoreType.DMA,
            pltpu.SemaphoreType.DMA,
        ],
    )


def kernel(x, var, label_mat):
    del var
    neg_dis, idx = _distances_and_argmin(x, label_mat)
    quantized = _sc_gather()(label_mat, idx)
    return quantized, neg_dis
